# Initial kernel scaffold; baseline (speedup 1.0000x reference)
import jax
import jax.numpy as jnp
from jax.experimental import pallas as pl


def kernel(imgs):
    raise NotImplementedError("write your pallas kernel here")


# speedup vs baseline: 15.9769x; 15.9769x over previous
import jax
import jax.numpy as jnp
import numpy as np
from jax.experimental import pallas as pl
from jax.experimental.pallas import tpu as pltpu

_B, _C, _H, _W = 8, 96, 224, 224
_S = 32
_NR = 4

_PLAN = (
    ((139, 106, 100, 73), (64, 38, 100, 73), (121, 78, 64, 38), (100, 73, 139, 106)),
    ((5, 25, 177, 101), (177, 101, 12, 186), (128, 175, 177, 101), (12, 186, 128, 175)),
    ((163, 100, 148, 81), (148, 81, 163, 100), (62, 137, 77, 23), (77, 23, 148, 81)),
    ((157, 143, 136, 175), (37, 93, 76, 118), (136, 175, 76, 118), (76, 118, 157, 143)),
    ((79, 91, 153, 180), (153, 180, 118, 5), (118, 5, 153, 180), (73, 126, 153, 180)),
    ((141, 100, 12, 18), (42, 23, 146, 125), (12, 18, 146, 125), (146, 125, 12, 18)),
    ((137, 162, 178, 76), (120, 54, 167, 86), (178, 76, 137, 162), (167, 86, 178, 76)),
    ((31, 69, 13, 54), (62, 63, 148, 175), (13, 54, 148, 175), (148, 175, 13, 54)),
)


_CB = 4


def _swap_body(in_ref, out_ref):
    out_ref[...] = in_ref[...]
    for b in range(_B):
        for (yd, xd, ys, xs) in _PLAN[b]:
            out_ref[b, :, yd:yd + _S, xd:xd + _S] = in_ref[b, :, ys:ys + _S, xs:xs + _S]


def kernel(imgs):
    return pl.pallas_call(
        _swap_body,
        grid=(_C // _CB,),
        in_specs=[pl.BlockSpec((_B, _CB, _H, _W), lambda c: (0, c, 0, 0))],
        out_specs=pl.BlockSpec((_B, _CB, _H, _W), lambda c: (0, c, 0, 0)),
        out_shape=jax.ShapeDtypeStruct((_B, _C, _H, _W), imgs.dtype),
    )(imgs)
